# baseline (speedup 1.0000x reference)
import functools

import jax
import jax.numpy as jnp
from jax import lax
from jax.experimental import pallas as pl
from jax.experimental.pallas import tpu as pltpu
from jax.experimental.pallas import tpu_sc as plsc

NC = 2
NS = 16
NW = NC * NS
B = 8192
H = 128
GYD = 512
CELLS = 512 * 512
SEG = CELLS // NW
PTS = B // NW
K = 25
NCH = PTS * K // 128
PB = 64
PBN = PTS * K // PB + 2

_mesh = plsc.VectorSubcoreMesh(core_axis_name="c", subcore_axis_name="s")
_params = pltpu.CompilerParams(needs_layout_passes=False,
                               use_tc_tiling_on_sc=True)


def _wid():
    return lax.axis_index("s") * NC + lax.axis_index("c")


@functools.partial(
    pl.kernel,
    out_type=(
        jax.ShapeDtypeStruct((CELLS,), jnp.int32),
        jax.ShapeDtypeStruct((B,), jnp.int32),
    ),
    mesh=_mesh,
    compiler_params=_params,
    scratch_types=[
        pltpu.VMEM((2 * B,), jnp.int32),
        pltpu.VMEM((SEG,), jnp.int32),
        pltpu.VMEM((B,), jnp.int32),
    ],
)
def _build(gi_hbm, owner_hbm, gpk_hbm, giv, ownv, gpkv):
    wid = _wid()
    seg0 = wid * SEG

    pltpu.sync_copy(gi_hbm, giv)

    neg1 = jnp.full((16,), -1, jnp.int32)

    def init_body(i, carry):
        ownv[pl.ds(i * 16, 16)] = neg1
        return carry

    lax.fori_loop(0, SEG // 16, init_body, 0)

    iota = lax.iota(jnp.int32, 16)

    def scan_body(v, carry):
        b0 = v * 16
        pvec = 2 * (b0 + iota)
        gxc = plsc.load_gather(giv, [pvec])
        gyc = plsc.load_gather(giv, [pvec + 1])
        flat = gxc * GYD + gyc
        gpkv[pl.ds(b0, 16)] = flat
        _, last = plsc.scan_count(flat)
        local = flat - seg0
        inr = (local >= 0) & (local < SEG)
        lc = jnp.clip(local, 0, SEG - 1)
        plsc.store_scatter(ownv, [lc], b0 + iota, mask=last & inr)
        return carry

    lax.fori_loop(0, B // 16, scan_body, 0)

    pltpu.sync_copy(ownv, owner_hbm.at[pl.ds(seg0, SEG)])
    pltpu.sync_copy(gpkv.at[pl.ds(wid * PTS, PTS)],
                    gpk_hbm.at[pl.ds(wid * PTS, PTS)])


@functools.partial(
    pl.kernel,
    out_type=jax.ShapeDtypeStruct((B * K, H), jnp.float32),
    mesh=_mesh,
    compiler_params=_params,
    scratch_types=[
        pltpu.VMEM((B,), jnp.int32),
        pltpu.VMEM((NCH, 128), jnp.int32),
        pltpu.VMEM((8, 128), jnp.int32),
        pltpu.VMEM((PBN, PB), jnp.int32),
        pltpu.VMEM((PBN, PB), jnp.int32),
        pltpu.VMEM((4, 128, H), jnp.float32),
        pltpu.VMEM((2, PB, H), jnp.float32),
        pltpu.SemaphoreType.DMA((8,)),
        pltpu.SemaphoreType.DMA((4,)),
        pltpu.SemaphoreType.DMA((4,)),
        pltpu.SemaphoreType.DMA,
        pltpu.SemaphoreType.DMA((2,)),
    ],
)
def _gather(gpk_hbm, owner_hbm, mem_hbm, upd_hbm, out_hbm,
            giv, cellidx, ownb, pidx, pown, rows, ubuf,
            semO, semR, semW, semU, semP):
    wid = _wid()
    row0 = wid * (K * PTS)
    pltpu.sync_copy(gpk_hbm, giv)

    iota = lax.iota(jnp.int32, 16)

    def pre_body(t, carry):
        p = row0 + t * 16 + iota
        n = 25 * (p & (B - 1)) + (p >> 13)
        k0 = n >> 13
        b0 = n & (B - 1)
        q = k0 // 5
        di = q - 2
        dj = k0 - 5 * q - 2
        f = plsc.load_gather(giv, [b0])
        cx = jnp.maximum((f >> 9) + di, 0)
        cy = jnp.maximum((f & (GYD - 1)) + dj, 0)
        cellidx[t >> 3, pl.ds((t & 7) * 16, 16)] = cx * GYD + cy
        return carry

    lax.fori_loop(0, K * PTS // 16, pre_body, 0)

    def fire_own(j):
        sl = j % 8
        pltpu.async_copy(owner_hbm.at[cellidx.at[j]], ownb.at[sl], semO.at[sl])

    for j0 in range(8):
        fire_own(j0)

    def issue(j):
        s = j % 4
        pltpu.async_copy(mem_hbm.at[cellidx.at[j]], rows.at[s], semR.at[s])

    issue(0)
    issue(1)

    def chunk_body(j, cnt):
        s = j % 4
        sl = j % 8

        pltpu.make_async_copy(
            owner_hbm.at[pl.ds(0, 128)], ownb.at[sl], semO.at[sl]).wait()
        for u in range(8):
            o = ownb[sl, pl.ds(u * 16, 16)]
            m = o >= 0
            dest = cnt + plsc.cumsum(m.astype(jnp.int32)) - 1
            orow = row0 + j * 128 + u * 16 + iota
            plsc.store_scatter(pidx, [dest >> 6, dest & (PB - 1)], orow,
                               mask=m)
            plsc.store_scatter(pown, [dest >> 6, dest & (PB - 1)], o, mask=m)
            cnt = cnt + plsc.all_reduce_population_count(m)[0]

        @pl.when(j + 8 < NCH)
        def _refire():
            fire_own(j + 8)

        pltpu.make_async_copy(
            mem_hbm.at[pl.ds(0, 128)], rows.at[s], semR.at[s]).wait()
        out0 = row0 + j * 128
        pltpu.async_copy(rows.at[s], out_hbm.at[pl.ds(out0, 128)], semW.at[s])

        @pl.when(j + 2 < NCH)
        def _prefetch():
            s2 = (j + 2) % 4

            @pl.when(j >= 2)
            def _drain_out():
                pltpu.make_async_copy(
                    rows.at[s2], out_hbm.at[pl.ds(0, 128)], semW.at[s2]).wait()

            issue(j + 2)

        return cnt

    n = lax.fori_loop(0, NCH, chunk_body, 0)

    for jlast in (NCH - 2, NCH - 1):
        s2 = jlast % 4
        pltpu.make_async_copy(
            rows.at[s2], out_hbm.at[pl.ds(0, 128)], semW.at[s2]).wait()

    @pl.when(n > 0)
    def _patch():
        pos0 = pidx[0, pl.ds(0, 16)][0]
        own0 = pown[0, pl.ds(0, 16)][0]

        def pad_body(v, carry):
            dest = n + v * 16 + iota
            m = dest < PBN * PB
            dc = jnp.minimum(dest, PBN * PB - 1)
            plsc.store_scatter(pidx, [dc >> 6, dc & (PB - 1)],
                               jnp.zeros((16,), jnp.int32) + pos0, mask=m)
            plsc.store_scatter(pown, [dc >> 6, dc & (PB - 1)],
                               jnp.zeros((16,), jnp.int32) + own0, mask=m)
            return carry

        lax.fori_loop(0, PB // 16, pad_body, 0)

        nb = (n + PB - 1) // PB

        def patch_body(bi, carry):
            sl = bi % 2

            @pl.when(bi >= 2)
            def _drain_scatter():
                pltpu.make_async_copy(
                    ubuf.at[sl], out_hbm.at[pl.ds(0, PB)], semP.at[sl]).wait()

            pltpu.async_copy(upd_hbm.at[pown.at[bi]], ubuf.at[sl], semU).wait()
            pltpu.async_copy(ubuf.at[sl], out_hbm.at[pidx.at[bi]], semP.at[sl])
            return carry

        lax.fori_loop(0, nb, patch_body, 0)

        def drain_body(bi, carry):
            pltpu.make_async_copy(
                ubuf.at[bi % 2], out_hbm.at[pl.ds(0, PB)], semP.at[bi % 2]).wait()
            return carry

        lax.fori_loop(jnp.maximum(nb - 2, 0), nb, drain_body, 0)


def kernel(grid_input, updates, memory):
    gi = grid_input.reshape(2 * B)
    memflat = memory.reshape(CELLS, H)
    owner, gpk = _build(gi)
    outflat = _gather(gpk, owner, memflat, updates)
    return outflat.reshape(K, B, H).transpose(1, 0, 2)
